# baseline (speedup 1.0000x reference)
import functools

import jax
import jax.numpy as jnp
from jax import lax
from jax.experimental import pallas as pl
from jax.experimental.pallas import tpu as pltpu
from jax.experimental.pallas import tpu_sc as plsc

SITE_EMBED_DIM = 64

_NC = 2
_NS = 16
_NW = _NC * _NS

_C = 128
_G = 4


def _gather_kernel(
    n_super, d, idx_hbm, table_hbm, out_hbm, idx_v, table_v, rows_v, g0, g1, w0, w1
):
    wid = lax.axis_index("s") * _NC + lax.axis_index("c")
    base = wid * (n_super * _G * _C)
    n_chunks = n_super * _G

    sid = lax.axis_index("s")
    @pl.when(sid == 0)
    def _():
        pltpu.sync_copy(table_hbm, table_v)
    pltpu.sync_copy(idx_hbm.at[wid], idx_v)
    plsc.subcore_barrier()

    def gather(t, slot, gsem):
        pltpu.async_copy(
            table_v.at[idx_v.at[pl.ds(t * _G * _C, _G * _C)]], rows_v.at[slot], gsem
        )

    def gather_wait(slot, gsem):
        pltpu.make_async_copy(
            table_v.at[idx_v.at[pl.ds(0, _G * _C)]], rows_v.at[slot], gsem
        ).wait()

    def write(t, slot, wsem):
        pltpu.async_copy(
            rows_v.at[slot],
            out_hbm.at[pl.ds(base + t * _G * _C, _G * _C), pl.ds(0, d)],
            wsem,
        )

    def write_wait(slot, wsem):
        pltpu.make_async_copy(
            rows_v.at[slot], out_hbm.at[pl.ds(base, _G * _C), pl.ds(0, d)], wsem
        ).wait()

    gather(0, 0, g0)
    gather(1, 1, g1)

    def body(tt, carry):
        t0 = 2 * tt
        t1 = t0 + 1
        gather_wait(0, g0)
        write(t0, 0, w0)
        write_wait(0, w0)
        gather(t0 + 2, 0, g0)
        gather_wait(1, g1)
        write(t1, 1, w1)
        write_wait(1, w1)
        gather(t1 + 2, 1, g1)
        return carry

    lax.fori_loop(0, n_super // 2 - 1, body, 0)

    t0 = n_super - 2
    t1 = n_super - 1
    gather_wait(0, g0)
    write(t0, 0, w0)
    gather_wait(1, g1)
    write(t1, 1, w1)
    write_wait(0, w0)
    write_wait(1, w1)


def kernel(x, embed_site):
    n_rows, n_cols = x.shape
    d = embed_site.shape[1]
    total = n_rows * n_cols
    assert total % (_NW * _C * _G) == 0
    n_super = total // (_NW * _C * _G)
    n_chunks = n_super * _G

    idx = x.reshape(_NW, n_chunks * _C).astype(jnp.int32)

    mesh = plsc.VectorSubcoreMesh(
        core_axis_name="c", subcore_axis_name="s", num_cores=_NC, num_subcores=_NS
    )
    run = pl.kernel(
        functools.partial(_gather_kernel, n_super, d),
        out_type=jax.ShapeDtypeStruct((total, 2 * d), jnp.float32),
        mesh=mesh,
        scratch_types=[
            pltpu.VMEM((n_chunks * _C,), jnp.int32),
            pltpu.VMEM_SHARED(embed_site.shape, jnp.float32),
            pltpu.VMEM((2, _G * _C, d), jnp.float32),
            pltpu.SemaphoreType.DMA,
            pltpu.SemaphoreType.DMA,
            pltpu.SemaphoreType.DMA,
            pltpu.SemaphoreType.DMA,
        ],
        compiler_params=pltpu.CompilerParams(use_tc_tiling_on_sc=False),
    )
    return run(idx, embed_site)[:, :d]


# speedup vs baseline: 1.0084x; 1.0084x over previous
import functools

import jax
import jax.numpy as jnp
from jax import lax
from jax.experimental import pallas as pl
from jax.experimental.pallas import tpu as pltpu
from jax.experimental.pallas import tpu_sc as plsc

SITE_EMBED_DIM = 64

_NC = 2
_NS = 16
_NW = _NC * _NS

_B = 256
_NSLOT = 4


def _gather_kernel(
    n_chunks, d, idx_hbm, table_hbm, out_hbm, idx_v, table_v, rows_v, gsems, wsems
):
    wid = lax.axis_index("s") * _NC + lax.axis_index("c")
    base = wid * (n_chunks * _B)

    sid = lax.axis_index("s")

    @pl.when(sid == 0)
    def _():
        pltpu.sync_copy(table_hbm, table_v)

    pltpu.sync_copy(idx_hbm.at[wid], idx_v)
    plsc.subcore_barrier()

    def gather(u, slot):
        pltpu.async_copy(
            table_v.at[idx_v.at[pl.ds(u * _B, _B)]], rows_v.at[slot], gsems[slot]
        )

    def gather_wait(slot):
        pltpu.make_async_copy(
            table_v.at[idx_v.at[pl.ds(0, _B)]], rows_v.at[slot], gsems[slot]
        ).wait()

    def write(u, slot):
        pltpu.async_copy(
            rows_v.at[slot],
            out_hbm.at[pl.ds(base + u * _B, _B), pl.ds(0, d)],
            wsems[slot],
        )

    def write_wait(slot):
        pltpu.make_async_copy(
            rows_v.at[slot], out_hbm.at[pl.ds(base, _B), pl.ds(0, d)], wsems[slot]
        ).wait()

    gather(0, 0)
    gather(1, 1)
    gather_wait(0)
    write(0, 0)
    gather(2, 2)
    gather_wait(1)
    write(1, 1)
    gather(3, 3)

    def body(k, carry):
        u0 = 2 + 4 * k
        for j in range(4):
            u = u0 + j
            s = (2 + j) % _NSLOT
            sr = j % _NSLOT
            gather_wait(s)
            write(u, s)
            write_wait(sr)
            gather(u + 2, sr)
        return carry

    lax.fori_loop(0, (n_chunks - 4) // 4, body, 0)

    gather_wait(2)
    write(n_chunks - 2, 2)
    gather_wait(3)
    write(n_chunks - 1, 3)
    write_wait(0)
    write_wait(1)
    write_wait(2)
    write_wait(3)


def kernel(x, embed_site):
    n_rows, n_cols = x.shape
    d = embed_site.shape[1]
    total = n_rows * n_cols
    assert total % (_NW * _B) == 0
    n_chunks = total // (_NW * _B)
    assert (n_chunks - 4) % 4 == 0

    idx = x.reshape(_NW, n_chunks * _B).astype(jnp.int32)

    mesh = plsc.VectorSubcoreMesh(
        core_axis_name="c", subcore_axis_name="s", num_cores=_NC, num_subcores=_NS
    )
    run = pl.kernel(
        functools.partial(_gather_kernel, n_chunks, d),
        out_type=jax.ShapeDtypeStruct((total, 2 * d), jnp.float32),
        mesh=mesh,
        scratch_types=[
            pltpu.VMEM((n_chunks * _B,), jnp.int32),
            pltpu.VMEM_SHARED(embed_site.shape, jnp.float32),
            pltpu.VMEM((_NSLOT, _B, d), jnp.float32),
            [pltpu.SemaphoreType.DMA] * _NSLOT,
            [pltpu.SemaphoreType.DMA] * _NSLOT,
        ],
        compiler_params=pltpu.CompilerParams(use_tc_tiling_on_sc=False),
    )
    return run(idx, embed_site)[:, :d]
